# Initial kernel scaffold; baseline (speedup 1.0000x reference)
import jax
import jax.numpy as jnp
from jax.experimental import pallas as pl


def kernel(raw_scores, cluster_sizes):
    raise NotImplementedError("write your pallas kernel here")


# speedup vs baseline: 6.9518x; 6.9518x over previous
import functools

import jax
import jax.numpy as jnp
from jax.experimental import pallas as pl


_BS = 8
_N = 2048
_NC = 32
_ROWS = 256


def _mse_kernel(x_ref, prev_ref, ends_ref, out_ref):
    b = pl.program_id(0)
    i = pl.program_id(1)

    prev = prev_ref[0, 0, :].astype(jnp.int32).reshape(1, _NC)
    ends = ends_ref[0, 0, :].astype(jnp.int32).reshape(1, _NC)

    row0 = i * _ROWS
    rows = jax.lax.broadcasted_iota(jnp.int32, (_ROWS, _NC), 0) + row0
    mr = ((rows >= prev) & (rows < ends)).astype(jnp.float32)

    cols = jax.lax.broadcasted_iota(jnp.int32, (_NC, _N), 1)
    mct = ((cols >= prev.reshape(_NC, 1)) & (cols < ends.reshape(_NC, 1))).astype(
        jnp.float32
    )

    overlap = jax.lax.dot_general(
        mr, mct, (((1,), (0,)), ((), ())), preferred_element_type=jnp.float32
    )
    mask = (overlap > 0.5).astype(jnp.float32)

    diff = x_ref[0] - mask
    partial = jnp.sum(diff * diff)

    @pl.when((b == 0) & (i == 0))
    def _init():
        out_ref[...] = jnp.zeros((1, 1), jnp.float32)

    out_ref[...] += partial.reshape(1, 1)


@functools.partial(jax.jit, static_argnames=())
def _loss(raw_scores, prev, ends):
    n_blocks = _N // _ROWS
    total = pl.pallas_call(
        _mse_kernel,
        grid=(_BS, n_blocks),
        in_specs=[
            pl.BlockSpec((1, _ROWS, _N), lambda b, i: (b, i, 0)),
            pl.BlockSpec((1, 1, _NC), lambda b, i: (b, 0, 0)),
            pl.BlockSpec((1, 1, _NC), lambda b, i: (b, 0, 0)),
        ],
        out_specs=pl.BlockSpec((1, 1), lambda b, i: (0, 0)),
        out_shape=jax.ShapeDtypeStruct((1, 1), jnp.float32),
    )(raw_scores, prev, ends)
    return total[0, 0] / jnp.float32(_BS * _N * _N)


def kernel(raw_scores, cluster_sizes):
    cs = cluster_sizes.astype(jnp.int32)
    prev = jnp.concatenate(
        [jnp.zeros((_BS, 1), dtype=jnp.int32), cs[:, :-1]], axis=1
    ).reshape(_BS, 1, _NC)
    ends = prev + cs.reshape(_BS, 1, _NC)
    return _loss(raw_scores, prev, ends)


# speedup vs baseline: 8.6530x; 1.2447x over previous
import functools

import jax
import jax.numpy as jnp
from jax.experimental import pallas as pl


_BS = 8
_N = 2048
_NC = 32
_ROWS = 512


def _mse_kernel(x_ref, starts_ref, ends_ref, out_ref):
    b = pl.program_id(0)
    i = pl.program_id(1)

    starts = starts_ref[0, 0, :].reshape(1, _NC)
    ends = ends_ref[0, 0, :].reshape(1, _NC)

    row0 = i * _ROWS
    rows = jax.lax.broadcasted_iota(jnp.int32, (_ROWS, _NC), 0) + row0
    inb = (rows >= starts) & (rows < ends)
    lo = jnp.min(jnp.where(inb, starts, _N), axis=1, keepdims=True)
    hi = jnp.max(jnp.where(inb, ends, 0), axis=1, keepdims=True)

    cols = jax.lax.broadcasted_iota(jnp.int32, (_ROWS, _N), 1)
    pred = (cols >= lo) & (cols < hi)

    x = x_ref[0]
    diff = jnp.where(pred, x - 1.0, x)
    partial = jnp.sum(diff * diff)

    @pl.when((b == 0) & (i == 0))
    def _init():
        out_ref[...] = jnp.zeros((1, 1), jnp.float32)

    out_ref[...] += partial.reshape(1, 1)


@functools.partial(jax.jit, static_argnames=())
def _loss(raw_scores, starts, ends):
    n_blocks = _N // _ROWS
    total = pl.pallas_call(
        _mse_kernel,
        grid=(_BS, n_blocks),
        in_specs=[
            pl.BlockSpec((1, _ROWS, _N), lambda b, i: (b, i, 0)),
            pl.BlockSpec((1, 1, _NC), lambda b, i: (b, 0, 0)),
            pl.BlockSpec((1, 1, _NC), lambda b, i: (b, 0, 0)),
        ],
        out_specs=pl.BlockSpec((1, 1), lambda b, i: (0, 0)),
        out_shape=jax.ShapeDtypeStruct((1, 1), jnp.float32),
    )(raw_scores, starts, ends)
    return total[0, 0] / jnp.float32(_BS * _N * _N)


def kernel(raw_scores, cluster_sizes):
    cs = cluster_sizes.astype(jnp.int32)
    starts = jnp.concatenate(
        [jnp.zeros((_BS, 1), dtype=jnp.int32), cs[:, :-1]], axis=1
    ).reshape(_BS, 1, _NC)
    ends = starts + cs.reshape(_BS, 1, _NC)
    return _loss(raw_scores, starts, ends)


# speedup vs baseline: 10.0785x; 1.1647x over previous
import functools

import jax
import jax.numpy as jnp
from jax.experimental import pallas as pl
from jax.experimental.pallas import tpu as pltpu


_BS = 8
_N = 2048
_NC = 32
_ROWS = 512
_NBLK = _N // _ROWS


def _mse_kernel(x_ref, starts_ref, ends_ref, out_ref, acc_ref):
    b = pl.program_id(0)
    i = pl.program_id(1)

    @pl.when((b == 0) & (i == 0))
    def _init():
        acc_ref[...] = jnp.zeros_like(acc_ref)

    starts = starts_ref[0, 0, :].reshape(1, _NC)
    ends = ends_ref[0, 0, :].reshape(1, _NC)

    row0 = i * _ROWS
    rows = jax.lax.broadcasted_iota(jnp.int32, (_ROWS, _NC), 0) + row0
    inb = (rows >= starts) & (rows < ends)
    lo = jnp.min(jnp.where(inb, starts, _N), axis=1, keepdims=True)
    hi = jnp.max(jnp.where(inb, ends, 0), axis=1, keepdims=True)

    cols = jax.lax.broadcasted_iota(jnp.int32, (_ROWS, _N), 1)
    rel = jax.lax.bitcast_convert_type(cols - lo, jnp.uint32)
    width = jax.lax.bitcast_convert_type(hi - lo, jnp.uint32)
    pred = rel < width

    x = x_ref[0]
    diff = jnp.where(pred, x - 1.0, x)
    d2 = diff * diff
    acc_ref[...] += d2.reshape(_ROWS // 8, 8, _N).sum(axis=0)

    @pl.when((b == _BS - 1) & (i == _NBLK - 1))
    def _fin():
        out_ref[...] = jnp.sum(acc_ref[...]).reshape(1, 1)


@functools.partial(jax.jit, static_argnames=())
def _loss(raw_scores, starts, ends):
    total = pl.pallas_call(
        _mse_kernel,
        grid=(_BS, _NBLK),
        in_specs=[
            pl.BlockSpec((1, _ROWS, _N), lambda b, i: (b, i, 0)),
            pl.BlockSpec((1, 1, _NC), lambda b, i: (b, 0, 0)),
            pl.BlockSpec((1, 1, _NC), lambda b, i: (b, 0, 0)),
        ],
        out_specs=pl.BlockSpec((1, 1), lambda b, i: (0, 0)),
        out_shape=jax.ShapeDtypeStruct((1, 1), jnp.float32),
        scratch_shapes=[pltpu.VMEM((8, _N), jnp.float32)],
    )(raw_scores, starts, ends)
    return total[0, 0] / jnp.float32(_BS * _N * _N)


def kernel(raw_scores, cluster_sizes):
    cs = cluster_sizes.astype(jnp.int32)
    starts = jnp.concatenate(
        [jnp.zeros((_BS, 1), dtype=jnp.int32), cs[:, :-1]], axis=1
    ).reshape(_BS, 1, _NC)
    ends = starts + cs.reshape(_BS, 1, _NC)
    return _loss(raw_scores, starts, ends)


# speedup vs baseline: 10.5571x; 1.0475x over previous
import functools

import jax
import jax.numpy as jnp
from jax.experimental import pallas as pl
from jax.experimental.pallas import tpu as pltpu


_BS = 8
_N = 2048
_NC = 32
_ROWS = 512
_NBLK = _N // _ROWS


def _mse_kernel(x_ref, starts_ref, ends_ref, out_ref, acc_ref):
    b = pl.program_id(0)
    i = pl.program_id(1)

    @pl.when((b == 0) & (i == 0))
    def _init():
        acc_ref[...] = jnp.zeros_like(acc_ref)

    starts = starts_ref[0, 0, :].reshape(1, _NC)
    ends = ends_ref[0, 0, :].reshape(1, _NC)

    row0 = i * _ROWS
    rows = jax.lax.broadcasted_iota(jnp.int32, (_ROWS, _NC), 0) + row0
    inb = (rows >= starts) & (rows < ends)
    lo = jnp.min(jnp.where(inb, starts, _N), axis=1, keepdims=True)
    hi = jnp.max(jnp.where(inb, ends, 0), axis=1, keepdims=True)

    cols = jax.lax.broadcasted_iota(jnp.int32, (_ROWS, _N), 1)
    rel = jax.lax.bitcast_convert_type(cols - lo, jnp.uint32)
    width = jax.lax.bitcast_convert_type(hi - lo, jnp.uint32)
    pred = rel < width

    x = x_ref[0]
    diff = jnp.where(pred, x - 1.0, x)
    d2 = diff * diff
    ones = jnp.ones((8, _ROWS), jnp.float32)
    acc_ref[...] += jax.lax.dot_general(
        ones, d2, (((1,), (0,)), ((), ())), preferred_element_type=jnp.float32
    )

    @pl.when((b == _BS - 1) & (i == _NBLK - 1))
    def _fin():
        out_ref[...] = jnp.sum(acc_ref[...]).reshape(1, 1)


@functools.partial(jax.jit, static_argnames=())
def _loss(raw_scores, starts, ends):
    total = pl.pallas_call(
        _mse_kernel,
        grid=(_BS, _NBLK),
        in_specs=[
            pl.BlockSpec((1, _ROWS, _N), lambda b, i: (b, i, 0)),
            pl.BlockSpec((1, 1, _NC), lambda b, i: (b, 0, 0)),
            pl.BlockSpec((1, 1, _NC), lambda b, i: (b, 0, 0)),
        ],
        out_specs=pl.BlockSpec((1, 1), lambda b, i: (0, 0)),
        out_shape=jax.ShapeDtypeStruct((1, 1), jnp.float32),
        scratch_shapes=[pltpu.VMEM((8, _N), jnp.float32)],
    )(raw_scores, starts, ends)
    return total[0, 0] / jnp.float32(_BS * _N * _N * 8)


def kernel(raw_scores, cluster_sizes):
    cs = cluster_sizes.astype(jnp.int32)
    starts = jnp.concatenate(
        [jnp.zeros((_BS, 1), dtype=jnp.int32), cs[:, :-1]], axis=1
    ).reshape(_BS, 1, _NC)
    ends = starts + cs.reshape(_BS, 1, _NC)
    return _loss(raw_scores, starts, ends)


# speedup vs baseline: 12.2222x; 1.1577x over previous
import functools

import jax
import jax.numpy as jnp
from jax.experimental import pallas as pl
from jax.experimental.pallas import tpu as pltpu


_BS = 8
_N = 2048
_NC = 32
_ROWS = 1024
_NBLK = _N // _ROWS


def _mse_kernel(x_ref, starts_ref, ends_ref, out_ref, acc_ref):
    b = pl.program_id(0)
    i = pl.program_id(1)

    @pl.when((b == 0) & (i == 0))
    def _init():
        acc_ref[...] = jnp.zeros_like(acc_ref)

    starts = starts_ref[0, 0, :].reshape(1, _NC)
    ends = ends_ref[0, 0, :].reshape(1, _NC)

    row0 = i * _ROWS
    rows = jax.lax.broadcasted_iota(jnp.int32, (_ROWS, _NC), 0) + row0
    inb = (rows >= starts) & (rows < ends)
    lo = jnp.min(jnp.where(inb, starts, _N), axis=1, keepdims=True)
    hi = jnp.max(jnp.where(inb, ends, 0), axis=1, keepdims=True)

    cols = jax.lax.broadcasted_iota(jnp.int32, (_ROWS, _N), 1)
    rel = jax.lax.bitcast_convert_type(cols - lo, jnp.uint32)
    width = jax.lax.bitcast_convert_type(hi - lo, jnp.uint32)
    pred = rel < width

    x = x_ref[0]
    diff = jnp.where(pred, x - 1.0, x)
    d2 = diff * diff
    ones = jnp.ones((8, _ROWS), jnp.float32)
    acc_ref[...] += jax.lax.dot_general(
        ones, d2, (((1,), (0,)), ((), ())), preferred_element_type=jnp.float32
    )

    @pl.when((b == _BS - 1) & (i == _NBLK - 1))
    def _fin():
        out_ref[...] = jnp.sum(acc_ref[...]).reshape(1, 1)


@functools.partial(jax.jit, static_argnames=())
def _loss(raw_scores, starts, ends):
    total = pl.pallas_call(
        _mse_kernel,
        grid=(_BS, _NBLK),
        in_specs=[
            pl.BlockSpec((1, _ROWS, _N), lambda b, i: (b, i, 0)),
            pl.BlockSpec((1, 1, _NC), lambda b, i: (b, 0, 0)),
            pl.BlockSpec((1, 1, _NC), lambda b, i: (b, 0, 0)),
        ],
        out_specs=pl.BlockSpec((1, 1), lambda b, i: (0, 0)),
        out_shape=jax.ShapeDtypeStruct((1, 1), jnp.float32),
        scratch_shapes=[pltpu.VMEM((8, _N), jnp.float32)],
    )(raw_scores, starts, ends)
    return total[0, 0] / jnp.float32(_BS * _N * _N * 8)


def kernel(raw_scores, cluster_sizes):
    cs = cluster_sizes.astype(jnp.int32)
    starts = jnp.concatenate(
        [jnp.zeros((_BS, 1), dtype=jnp.int32), cs[:, :-1]], axis=1
    ).reshape(_BS, 1, _NC)
    ends = starts + cs.reshape(_BS, 1, _NC)
    return _loss(raw_scores, starts, ends)


# speedup vs baseline: 12.9568x; 1.0601x over previous
import functools

import jax
import jax.numpy as jnp
from jax.experimental import pallas as pl
from jax.experimental.pallas import tpu as pltpu


_BS = 8
_N = 2048
_NC = 32
_ROWS = 2048
_NBLK = _N // _ROWS


def _mse_kernel(x_ref, starts_ref, ends_ref, out_ref, acc_ref):
    b = pl.program_id(0)
    i = pl.program_id(1)

    @pl.when((b == 0) & (i == 0))
    def _init():
        acc_ref[...] = jnp.zeros_like(acc_ref)

    starts = starts_ref[0, 0, :].reshape(1, _NC)
    ends = ends_ref[0, 0, :].reshape(1, _NC)

    row0 = i * _ROWS
    rows = jax.lax.broadcasted_iota(jnp.int32, (_ROWS, _NC), 0) + row0
    inb = (rows >= starts) & (rows < ends)
    lo = jnp.min(jnp.where(inb, starts, _N), axis=1, keepdims=True)
    hi = jnp.max(jnp.where(inb, ends, 0), axis=1, keepdims=True)

    cols = jax.lax.broadcasted_iota(jnp.int32, (_ROWS, _N), 1)
    rel = jax.lax.bitcast_convert_type(cols - lo, jnp.uint32)
    width = jax.lax.bitcast_convert_type(hi - lo, jnp.uint32)
    pred = rel < width

    x = x_ref[0]
    diff = jnp.where(pred, x - 1.0, x)
    d2 = diff * diff
    ones = jnp.ones((8, _ROWS), jnp.float32)
    acc_ref[...] += jax.lax.dot_general(
        ones, d2, (((1,), (0,)), ((), ())), preferred_element_type=jnp.float32
    )

    @pl.when((b == _BS - 1) & (i == _NBLK - 1))
    def _fin():
        out_ref[...] = jnp.sum(acc_ref[...]).reshape(1, 1)


@functools.partial(jax.jit, static_argnames=())
def _loss(raw_scores, starts, ends):
    total = pl.pallas_call(
        _mse_kernel,
        grid=(_BS, _NBLK),
        in_specs=[
            pl.BlockSpec((1, _ROWS, _N), lambda b, i: (b, i, 0)),
            pl.BlockSpec((1, 1, _NC), lambda b, i: (b, 0, 0)),
            pl.BlockSpec((1, 1, _NC), lambda b, i: (b, 0, 0)),
        ],
        out_specs=pl.BlockSpec((1, 1), lambda b, i: (0, 0)),
        out_shape=jax.ShapeDtypeStruct((1, 1), jnp.float32),
        scratch_shapes=[pltpu.VMEM((8, _N), jnp.float32)],
    )(raw_scores, starts, ends)
    return total[0, 0] / jnp.float32(_BS * _N * _N * 8)


def kernel(raw_scores, cluster_sizes):
    cs = cluster_sizes.astype(jnp.int32)
    starts = jnp.concatenate(
        [jnp.zeros((_BS, 1), dtype=jnp.int32), cs[:, :-1]], axis=1
    ).reshape(_BS, 1, _NC)
    ends = starts + cs.reshape(_BS, 1, _NC)
    return _loss(raw_scores, starts, ends)


# speedup vs baseline: 13.0199x; 1.0049x over previous
import functools

import jax
import jax.numpy as jnp
from jax.experimental import pallas as pl
from jax.experimental.pallas import tpu as pltpu


_BS = 8
_N = 2048
_NC = 32
_ROWS = 2048
_NBLK = _N // _ROWS
_HALF = _N // 2


def _mse_kernel(x1_ref, x2_ref, starts_ref, ends_ref, out_ref, acc_ref):
    b = pl.program_id(0)

    @pl.when(b == 0)
    def _init():
        acc_ref[...] = jnp.zeros_like(acc_ref)

    starts = starts_ref[0, 0, :].reshape(1, _NC)
    ends = ends_ref[0, 0, :].reshape(1, _NC)

    rows = jax.lax.broadcasted_iota(jnp.int32, (_ROWS, _NC), 0)
    inb = (rows >= starts) & (rows < ends)
    lo = jnp.min(jnp.where(inb, starts, _N), axis=1, keepdims=True)
    hi = jnp.max(jnp.where(inb, ends, 0), axis=1, keepdims=True)

    ones = jnp.ones((8, _ROWS), jnp.float32)

    def half(x_ref, base):
        cols = jax.lax.broadcasted_iota(jnp.int32, (_ROWS, _HALF), 1) + base
        rel = jax.lax.bitcast_convert_type(cols - lo, jnp.uint32)
        width = jax.lax.bitcast_convert_type(hi - lo, jnp.uint32)
        pred = rel < width
        x = x_ref[0]
        diff = jnp.where(pred, x - 1.0, x)
        d2 = diff * diff
        return jax.lax.dot_general(
            ones, d2, (((1,), (0,)), ((), ())), preferred_element_type=jnp.float32
        )

    acc_ref[:, :_HALF] += half(x1_ref, 0)
    acc_ref[:, _HALF:] += half(x2_ref, _HALF)

    @pl.when(b == _BS - 1)
    def _fin():
        out_ref[...] = jnp.sum(acc_ref[...]).reshape(1, 1)


@functools.partial(jax.jit, static_argnames=())
def _loss(raw_scores, starts, ends):
    total = pl.pallas_call(
        _mse_kernel,
        grid=(_BS,),
        in_specs=[
            pl.BlockSpec((1, _ROWS, _HALF), lambda b: (b, 0, 0)),
            pl.BlockSpec((1, _ROWS, _HALF), lambda b: (b, 0, 1)),
            pl.BlockSpec((1, 1, _NC), lambda b: (b, 0, 0)),
            pl.BlockSpec((1, 1, _NC), lambda b: (b, 0, 0)),
        ],
        out_specs=pl.BlockSpec((1, 1), lambda b: (0, 0)),
        out_shape=jax.ShapeDtypeStruct((1, 1), jnp.float32),
        scratch_shapes=[pltpu.VMEM((8, _N), jnp.float32)],
    )(raw_scores, raw_scores, starts, ends)
    return total[0, 0] / jnp.float32(_BS * _N * _N * 8)


def kernel(raw_scores, cluster_sizes):
    cs = cluster_sizes.astype(jnp.int32)
    starts = jnp.concatenate(
        [jnp.zeros((_BS, 1), dtype=jnp.int32), cs[:, :-1]], axis=1
    ).reshape(_BS, 1, _NC)
    ends = starts + cs.reshape(_BS, 1, _NC)
    return _loss(raw_scores, starts, ends)
